# baseline (speedup 1.0000x reference)
import functools

import jax
import jax.numpy as jnp
from jax import lax
from jax.experimental import pallas as pl
from jax.experimental.pallas import tpu as pltpu
from jax.experimental.pallas import tpu_sc as plsc

_NUM_CORES = 2
_NUM_SUBCORES = 16
_NW = _NUM_CORES * _NUM_SUBCORES
_NBUF = 2


@functools.lru_cache(maxsize=None)
def _build_gather(B, V, W, CH):
    per_w = B // _NW
    n_ch = per_w // CH
    outer = n_ch // _NBUF
    mesh = plsc.VectorSubcoreMesh(core_axis_name="c", subcore_axis_name="s")

    @functools.partial(
        pl.kernel,
        mesh=mesh,
        out_type=jax.ShapeDtypeStruct((B, W), jnp.int32),
        scratch_types=[
            pltpu.VMEM((per_w,), jnp.int32),
            pltpu.VMEM((_NBUF, CH, W), jnp.int32),
            pltpu.SemaphoreType.DMA((_NBUF,)),
        ],
        compiler_params=pltpu.CompilerParams(use_tc_tiling_on_sc=False),
    )
    def k(table_hbm, idx_hbm, out_hbm, idx_v, rows_v, gsem):
        wid = lax.axis_index("s") * _NUM_CORES + lax.axis_index("c")
        base = wid * per_w
        pltpu.sync_copy(idx_hbm.at[pl.ds(base, per_w)], idx_v)

        def fire(j, b):
            pltpu.async_copy(
                table_hbm.at[idx_v.at[pl.ds(j * CH, CH)]],
                rows_v.at[b],
                gsem.at[b],
            )

        def drain(b):
            pltpu.make_async_copy(
                table_hbm.at[idx_v.at[pl.ds(0, CH)]],
                rows_v.at[b],
                gsem.at[b],
            ).wait()

        for b in range(_NBUF):
            fire(b, b)

        def body(i, carry):
            for b in range(_NBUF):
                j = i * _NBUF + b
                drain(b)
                pltpu.sync_copy(rows_v.at[b], out_hbm.at[pl.ds(base + j * CH, CH)])

                @pl.when(j + _NBUF < n_ch)
                def _():
                    fire(j + _NBUF, b)

            return carry

        lax.fori_loop(0, outer, body, 0)

    return k


def kernel(input, weight):
    B = input.shape[0] * input.shape[1]
    V, D = weight.shape
    idx = input.reshape(B).astype(jnp.int32)
    table_i32 = lax.bitcast_convert_type(weight.reshape(V, D // 2, 2), jnp.int32)
    out_i32 = _build_gather(B, V, D // 2, 1600)(table_i32, idx)
    return out_i32


# speedup vs baseline: 1.2976x; 1.2976x over previous
import functools

import jax
import jax.numpy as jnp
from jax import lax
from jax.experimental import pallas as pl
from jax.experimental.pallas import tpu as pltpu
from jax.experimental.pallas import tpu_sc as plsc

_NUM_CORES = 2
_NUM_SUBCORES = 16
_NW = _NUM_CORES * _NUM_SUBCORES
_NBUF = 2
_L = 16


@functools.lru_cache(maxsize=None)
def _build_retile(V, D, R):
    W = D // 2
    per_w = V // _NW
    n_ch = per_w // R
    mesh = plsc.VectorSubcoreMesh(core_axis_name="c", subcore_axis_name="s")

    @functools.partial(
        pl.kernel,
        mesh=mesh,
        out_type=jax.ShapeDtypeStruct((V, W), jnp.int32),
        scratch_types=[
            pltpu.VMEM((R, D), jnp.bfloat16),
            pltpu.VMEM((R, W), jnp.int32),
        ],
        compiler_params=pltpu.CompilerParams(use_tc_tiling_on_sc=False, needs_layout_passes=False),
    )
    def k1(tbl_bf16, out_i32, buf_bf, buf_i32):
        wid = lax.axis_index("s") * _NUM_CORES + lax.axis_index("c")
        base = wid * per_w

        def body(j, carry):
            off = base + j * R
            pltpu.sync_copy(tbl_bf16.at[pl.ds(off, R)], buf_bf)

            def conv(t, c2):
                for h in range(W // _L):
                    x = buf_bf[t, pl.ds(h * 2 * _L, 2 * _L)]
                    buf_i32[t, pl.ds(h * _L, _L)] = plsc.bitcast(x, jnp.int32)
                return c2

            lax.fori_loop(0, R, conv, 0)
            pltpu.sync_copy(buf_i32, out_i32.at[pl.ds(off, R)])
            return carry

        lax.fori_loop(0, n_ch, body, 0)

    return k1


@functools.lru_cache(maxsize=None)
def _build_gather(B, V, D, CH):
    W = D // 2
    per_w = B // _NW
    n_ch = per_w // CH
    outer = n_ch // _NBUF
    mesh = plsc.VectorSubcoreMesh(core_axis_name="c", subcore_axis_name="s")

    @functools.partial(
        pl.kernel,
        mesh=mesh,
        out_type=jax.ShapeDtypeStruct((B, D), jnp.bfloat16),
        scratch_types=[
            pltpu.VMEM((per_w,), jnp.int32),
            pltpu.VMEM((_NBUF, CH, W), jnp.int32),
            pltpu.VMEM((CH, D), jnp.bfloat16),
            pltpu.SemaphoreType.DMA((_NBUF,)),
        ],
        compiler_params=pltpu.CompilerParams(use_tc_tiling_on_sc=False, needs_layout_passes=False),
    )
    def k2(table_hbm, idx_hbm, out_hbm, idx_v, rows_v, bf_v, gsem):
        wid = lax.axis_index("s") * _NUM_CORES + lax.axis_index("c")
        base = wid * per_w
        pltpu.sync_copy(idx_hbm.at[pl.ds(base, per_w)], idx_v)

        def fire(j, b):
            pltpu.async_copy(
                table_hbm.at[idx_v.at[pl.ds(j * CH, CH)]],
                rows_v.at[b],
                gsem.at[b],
            )

        def drain(b):
            pltpu.make_async_copy(
                table_hbm.at[idx_v.at[pl.ds(0, CH)]],
                rows_v.at[b],
                gsem.at[b],
            ).wait()

        for b in range(_NBUF):
            fire(b, b)

        def body(i, carry):
            for b in range(_NBUF):
                j = i * _NBUF + b
                drain(b)

                def conv(t, c2):
                    for h in range(W // _L):
                        x = rows_v[b, t, pl.ds(h * _L, _L)]
                        bf_v[t, pl.ds(h * 2 * _L, 2 * _L)] = plsc.bitcast(
                            x, jnp.bfloat16
                        )
                    return c2

                lax.fori_loop(0, CH, conv, 0)
                pltpu.sync_copy(bf_v, out_hbm.at[pl.ds(base + j * CH, CH)])

                @pl.when(j + _NBUF < n_ch)
                def _():
                    fire(j + _NBUF, b)

            return carry

        lax.fori_loop(0, outer, body, 0)

    return k2


def kernel(input, weight):
    B = input.shape[0] * input.shape[1]
    V, D = weight.shape
    idx = input.reshape(B).astype(jnp.int32)
    table_i32 = _build_retile(V, D, 1250)(weight)
    out = _build_gather(B, V, D, 800)(table_i32, idx)
    return out.reshape(input.shape + (D,))


# speedup vs baseline: 1.3761x; 1.0605x over previous
import functools

import jax
import jax.numpy as jnp
from jax import lax
from jax.experimental import pallas as pl
from jax.experimental.pallas import tpu as pltpu
from jax.experimental.pallas import tpu_sc as plsc

_NUM_CORES = 2
_NUM_SUBCORES = 16
_NW = _NUM_CORES * _NUM_SUBCORES
_NBUF = 2
_L = 16


@functools.lru_cache(maxsize=None)
def _build_retile(V, D, R):
    W = D // 2
    per_w = V // _NW
    n_ch = per_w // R
    mesh = plsc.VectorSubcoreMesh(core_axis_name="c", subcore_axis_name="s")

    @functools.partial(
        pl.kernel,
        mesh=mesh,
        out_type=jax.ShapeDtypeStruct((V, W), jnp.int32),
        scratch_types=[
            pltpu.VMEM((R, D), jnp.bfloat16),
            pltpu.VMEM((R, W), jnp.int32),
        ],
        compiler_params=pltpu.CompilerParams(use_tc_tiling_on_sc=False, needs_layout_passes=False),
    )
    def k1(tbl_bf16, out_i32, buf_bf, buf_i32):
        wid = lax.axis_index("s") * _NUM_CORES + lax.axis_index("c")
        base = wid * per_w

        def body(j, carry):
            off = base + j * R
            pltpu.sync_copy(tbl_bf16.at[pl.ds(off, R)], buf_bf)

            def conv(t, c2):
                for r in range(2):
                    for h in range(W // _L):
                        x = buf_bf[2 * t + r, pl.ds(h * 2 * _L, 2 * _L)]
                        buf_i32[2 * t + r, pl.ds(h * _L, _L)] = plsc.bitcast(
                            x, jnp.int32
                        )
                return c2

            lax.fori_loop(0, R // 2, conv, 0)
            pltpu.sync_copy(buf_i32, out_i32.at[pl.ds(off, R)])
            return carry

        lax.fori_loop(0, n_ch, body, 0)

    return k1


@functools.lru_cache(maxsize=None)
def _build_gather(B, V, D, CH):
    W = D // 2
    per_w = B // _NW
    n_ch = per_w // CH
    outer = n_ch // _NBUF
    mesh = plsc.VectorSubcoreMesh(core_axis_name="c", subcore_axis_name="s")

    @functools.partial(
        pl.kernel,
        mesh=mesh,
        out_type=jax.ShapeDtypeStruct((B, D), jnp.bfloat16),
        scratch_types=[
            pltpu.VMEM((per_w,), jnp.int32),
            pltpu.VMEM((_NBUF, CH, W), jnp.int32),
            pltpu.VMEM((CH, D), jnp.bfloat16),
            pltpu.SemaphoreType.DMA((_NBUF,)),
        ],
        compiler_params=pltpu.CompilerParams(use_tc_tiling_on_sc=False, needs_layout_passes=False),
    )
    def k2(table_hbm, idx_hbm, out_hbm, idx_v, rows_v, bf_v, gsem):
        wid = lax.axis_index("s") * _NUM_CORES + lax.axis_index("c")
        base = wid * per_w
        pltpu.sync_copy(idx_hbm.at[pl.ds(base, per_w)], idx_v)

        def fire(j, b):
            pltpu.async_copy(
                table_hbm.at[idx_v.at[pl.ds(j * CH, CH)]],
                rows_v.at[b],
                gsem.at[b],
            )

        def drain(b):
            pltpu.make_async_copy(
                table_hbm.at[idx_v.at[pl.ds(0, CH)]],
                rows_v.at[b],
                gsem.at[b],
            ).wait()

        for b in range(_NBUF):
            fire(b, b)

        def body(i, carry):
            for b in range(_NBUF):
                j = i * _NBUF + b
                drain(b)

                def conv(t, c2):
                    for r in range(2):
                        for h in range(W // _L):
                            x = rows_v[b, 2 * t + r, pl.ds(h * _L, _L)]
                            bf_v[2 * t + r, pl.ds(h * 2 * _L, 2 * _L)] = (
                                plsc.bitcast(x, jnp.bfloat16)
                            )
                    return c2

                lax.fori_loop(0, CH // 2, conv, 0)
                pltpu.sync_copy(bf_v, out_hbm.at[pl.ds(base + j * CH, CH)])

                @pl.when(j + _NBUF < n_ch)
                def _():
                    fire(j + _NBUF, b)

            return carry

        lax.fori_loop(0, outer, body, 0)

    return k2


def kernel(input, weight):
    B = input.shape[0] * input.shape[1]
    V, D = weight.shape
    idx = input.reshape(B).astype(jnp.int32)
    table_i32 = _build_retile(V, D, 1250)(weight)
    out = _build_gather(B, V, D, 800)(table_i32, idx)
    return out.reshape(input.shape + (D,))
